# baseline (speedup 1.0000x reference)
import functools

import jax
import jax.numpy as jnp
from jax import lax
from jax.experimental import pallas as pl
from jax.experimental.pallas import tpu as pltpu
from jax.experimental.pallas import tpu_sc as plsc

B = 16384
SC_B = 4096
TC_B = B - SC_B
N = 200
NCONT = 3
NW = 32
LANES = 16
GROUP = 128
CHAINS = GROUP // LANES
PASS_CHAINS = 4
ROWS_PER_W = SC_B // NW
N_GROUPS = ROWS_PER_W // GROUP
UNROLL = 10
TC_W = 2048


def _sev_body(x_hbm, out_hbm, buf_a, buf_b, outb, sem_a, sem_b):
    nc = 2
    wid = lax.axis_index("s") * nc + lax.axis_index("c")
    base_row = wid * ROWS_PER_W

    zeros = jnp.zeros((LANES,), jnp.float32)
    ones = jnp.ones((LANES,), jnp.float32)

    bufs = (buf_a, buf_b)
    sems = (sem_a, sem_b)

    def fetch(i):
        g, c = divmod(i, NCONT)
        return pltpu.async_copy(
            x_hbm.at[c, :, pl.ds(base_row + g * GROUP, GROUP)],
            bufs[i % 2], sems[i % 2])

    def dp_pass(buf, dp, h0):
        def body(t, carry):
            dp = [list(carry[5 * j:5 * j + 5]) for j in range(PASS_CHAINS)]
            for u in range(UNROLL):
                n = t * UNROLL + u
                for j in range(PASS_CHAINS):
                    d0, d1, d2, d3, d4 = dp[j]
                    pi = buf[n, pl.ds((h0 + j) * LANES, LANES)]
                    om = 1.0 - pi
                    dp[j] = [
                        d0 * om,
                        d1 * om + d0 * pi,
                        d2 * om + d1 * pi,
                        d3 * om + d2 * pi,
                        d4 * om + d3 * pi,
                    ]
            return tuple(v for chain in dp for v in chain)

        res = lax.fori_loop(0, N // UNROLL, body,
                            tuple(v for chain in dp for v in chain))
        return [list(res[5 * j:5 * j + 5]) for j in range(PASS_CHAINS)]

    handles = {0: fetch(0)}
    for g in range(N_GROUPS):
        dpA = [[ones, zeros, zeros, zeros, zeros] for _ in range(PASS_CHAINS)]
        dpB = [[ones, zeros, zeros, zeros, zeros] for _ in range(PASS_CHAINS)]
        for c in range(NCONT):
            i = g * NCONT + c
            if i + 1 < N_GROUPS * NCONT:
                handles[i + 1] = fetch(i + 1)
            handles.pop(i).wait()
            buf = bufs[i % 2]
            dpA = dp_pass(buf, dpA, 0)
            dpB = dp_pass(buf, dpB, PASS_CHAINS)

        for h in range(CHAINS):
            d0, d1, d2, d3, d4 = (dpA if h < PASS_CHAINS
                                  else dpB)[h % PASS_CHAINS]
            sev0 = d0
            sev1 = d1 + d2
            sev2 = d3 + d4
            sev3 = 1.0 - (sev0 + sev1 + sev2)
            col0 = g * GROUP + h * LANES
            for k, val in enumerate((sev0, sev1, sev2, sev3, zeros)):
                outb[k, pl.ds(col0, LANES)] = val

    pltpu.sync_copy(outb, out_hbm.at[:, pl.ds(base_row, ROWS_PER_W)])


def _tc_body(x_ref, o_ref):
    shape = (8, TC_W)
    ones = jnp.ones(shape, jnp.float32)
    zeros = jnp.zeros(shape, jnp.float32)
    dp = (ones, zeros, zeros, zeros, zeros)

    for c in range(NCONT):
        def body(t, carry):
            d0, d1, d2, d3, d4 = carry
            pi = x_ref[c, pl.ds(t * 8, 8), :]
            om = 1.0 - pi
            return (
                d0 * om,
                d1 * om + d0 * pi,
                d2 * om + d1 * pi,
                d3 * om + d2 * pi,
                d4 * om + d3 * pi,
            )

        dp = lax.fori_loop(0, N // 8, body, dp)

    d = list(dp)
    while d[0].shape[0] > 1:
        w = d[0].shape[0] // 2
        a = [v[:w] for v in d]
        b = [v[w:] for v in d]
        d = [
            a[0] * b[0],
            a[0] * b[1] + a[1] * b[0],
            a[0] * b[2] + a[1] * b[1] + a[2] * b[0],
            a[0] * b[3] + a[1] * b[2] + a[2] * b[1] + a[3] * b[0],
            a[0] * b[4] + a[1] * b[3] + a[2] * b[2] + a[3] * b[1]
            + a[4] * b[0],
        ]
    sev0 = d[0]
    sev1 = d[1] + d[2]
    sev2 = d[3] + d[4]
    sev3 = 1.0 - (sev0 + sev1 + sev2)
    sev4 = jnp.zeros((1, TC_W), jnp.float32)
    for k, val in enumerate((sev0, sev1, sev2, sev3, sev4)):
        o_ref[pl.ds(k, 1), :] = val


@jax.jit
def kernel(x):
    xt = jnp.transpose(x, (0, 2, 1))
    mesh = plsc.VectorSubcoreMesh(core_axis_name="c", subcore_axis_name="s")
    run = functools.partial(
        pl.kernel,
        mesh=mesh,
        out_type=jax.ShapeDtypeStruct((5, SC_B), jnp.float32),
        scratch_types=[
            pltpu.VMEM((N, GROUP), jnp.float32),
            pltpu.VMEM((N, GROUP), jnp.float32),
            pltpu.VMEM((5, ROWS_PER_W), jnp.float32),
            pltpu.SemaphoreType.DMA,
            pltpu.SemaphoreType.DMA,
        ],
        compiler_params=pltpu.CompilerParams(needs_layout_passes=False),
    )(_sev_body)
    sc_out = run(xt)

    tc_out = pl.pallas_call(
        _tc_body,
        grid=(TC_B // TC_W,),
        in_specs=[pl.BlockSpec(
            (NCONT, N, TC_W), lambda i: (0, 0, SC_B // TC_W + i))],
        out_specs=pl.BlockSpec((5, TC_W), lambda i: (0, i)),
        out_shape=jax.ShapeDtypeStruct((5, TC_B), jnp.float32),
    )(xt)

    out = jnp.concatenate([sc_out, tc_out], axis=1)
    return out.T


# speedup vs baseline: 1.3021x; 1.3021x over previous
import functools

import jax
import jax.numpy as jnp
from jax import lax
from jax.experimental import pallas as pl
from jax.experimental.pallas import tpu as pltpu
from jax.experimental.pallas import tpu_sc as plsc

B = 16384
SC_B = 4096
TC_B = B - SC_B
N = 200
NCONT = 3
NW = 32
LANES = 16
GROUP = 128
CHAINS = GROUP // LANES
PASS_CHAINS = 4
ROWS_PER_W = SC_B // NW
N_GROUPS = ROWS_PER_W // GROUP
UNROLL = 10
TC_W = 1024


def _sev_body(x_hbm, out_hbm, buf_a, buf_b, outb, sem_a, sem_b):
    nc = 2
    wid = lax.axis_index("s") * nc + lax.axis_index("c")
    base_row = wid * ROWS_PER_W

    zeros = jnp.zeros((LANES,), jnp.float32)
    ones = jnp.ones((LANES,), jnp.float32)

    bufs = (buf_a, buf_b)
    sems = (sem_a, sem_b)

    def fetch(i):
        g, c = divmod(i, NCONT)
        return pltpu.async_copy(
            x_hbm.at[c, :, pl.ds(base_row + g * GROUP, GROUP)],
            bufs[i % 2], sems[i % 2])

    def dp_pass(buf, dp, h0):
        def body(t, carry):
            dp = [list(carry[5 * j:5 * j + 5]) for j in range(PASS_CHAINS)]
            for u in range(UNROLL):
                n = t * UNROLL + u
                for j in range(PASS_CHAINS):
                    d0, d1, d2, d3, d4 = dp[j]
                    pi = buf[n, pl.ds((h0 + j) * LANES, LANES)]
                    om = 1.0 - pi
                    dp[j] = [
                        d0 * om,
                        d1 * om + d0 * pi,
                        d2 * om + d1 * pi,
                        d3 * om + d2 * pi,
                        d4 * om + d3 * pi,
                    ]
            return tuple(v for chain in dp for v in chain)

        res = lax.fori_loop(0, N // UNROLL, body,
                            tuple(v for chain in dp for v in chain))
        return [list(res[5 * j:5 * j + 5]) for j in range(PASS_CHAINS)]

    handles = {0: fetch(0)}
    for g in range(N_GROUPS):
        dpA = [[ones, zeros, zeros, zeros, zeros] for _ in range(PASS_CHAINS)]
        dpB = [[ones, zeros, zeros, zeros, zeros] for _ in range(PASS_CHAINS)]
        for c in range(NCONT):
            i = g * NCONT + c
            if i + 1 < N_GROUPS * NCONT:
                handles[i + 1] = fetch(i + 1)
            handles.pop(i).wait()
            buf = bufs[i % 2]
            dpA = dp_pass(buf, dpA, 0)
            dpB = dp_pass(buf, dpB, PASS_CHAINS)

        for h in range(CHAINS):
            d0, d1, d2, d3, d4 = (dpA if h < PASS_CHAINS
                                  else dpB)[h % PASS_CHAINS]
            sev0 = d0
            sev1 = d1 + d2
            sev2 = d3 + d4
            sev3 = 1.0 - (sev0 + sev1 + sev2)
            col0 = g * GROUP + h * LANES
            for k, val in enumerate((sev0, sev1, sev2, sev3, zeros)):
                outb[k, pl.ds(col0, LANES)] = val

    pltpu.sync_copy(outb, out_hbm.at[:, pl.ds(base_row, ROWS_PER_W)])


def _tc_body(x_ref, o_ref):
    shape = (8, TC_W)
    ones = jnp.ones(shape, jnp.float32)
    zeros = jnp.zeros(shape, jnp.float32)
    dp = (ones, zeros, zeros, zeros, zeros)

    for c in range(NCONT):
        def pair_body(t, carry):
            d0, d1, d2, d3, d4 = carry
            p = x_ref[c, pl.ds(t * 16, 8), :]
            q = x_ref[c, pl.ds(t * 16 + 8, 8), :]
            e2 = p * q
            s = p + q
            e1 = s - e2 - e2
            e0 = (1.0 - s) + e2
            return (
                d0 * e0,
                d1 * e0 + d0 * e1,
                d2 * e0 + d1 * e1 + d0 * e2,
                d3 * e0 + d2 * e1 + d1 * e2,
                d4 * e0 + d3 * e1 + d2 * e2,
            )

        dp = lax.fori_loop(0, N // 16, pair_body, dp)
        d0, d1, d2, d3, d4 = dp
        pi = x_ref[c, pl.ds(N - 8, 8), :]
        om = 1.0 - pi
        dp = (
            d0 * om,
            d1 * om + d0 * pi,
            d2 * om + d1 * pi,
            d3 * om + d2 * pi,
            d4 * om + d3 * pi,
        )

    d = list(dp)
    while d[0].shape[0] > 1:
        w = d[0].shape[0] // 2
        a = [v[:w] for v in d]
        b = [v[w:] for v in d]
        d = [
            a[0] * b[0],
            a[0] * b[1] + a[1] * b[0],
            a[0] * b[2] + a[1] * b[1] + a[2] * b[0],
            a[0] * b[3] + a[1] * b[2] + a[2] * b[1] + a[3] * b[0],
            a[0] * b[4] + a[1] * b[3] + a[2] * b[2] + a[3] * b[1]
            + a[4] * b[0],
        ]
    sev0 = d[0]
    sev1 = d[1] + d[2]
    sev2 = d[3] + d[4]
    sev3 = 1.0 - (sev0 + sev1 + sev2)
    sev4 = jnp.zeros((1, TC_W), jnp.float32)
    for k, val in enumerate((sev0, sev1, sev2, sev3, sev4)):
        o_ref[pl.ds(k, 1), :] = val


@jax.jit
def kernel(x):
    xt = jnp.transpose(x, (0, 2, 1))
    mesh = plsc.VectorSubcoreMesh(core_axis_name="c", subcore_axis_name="s")
    run = functools.partial(
        pl.kernel,
        mesh=mesh,
        out_type=jax.ShapeDtypeStruct((5, SC_B), jnp.float32),
        scratch_types=[
            pltpu.VMEM((N, GROUP), jnp.float32),
            pltpu.VMEM((N, GROUP), jnp.float32),
            pltpu.VMEM((5, ROWS_PER_W), jnp.float32),
            pltpu.SemaphoreType.DMA,
            pltpu.SemaphoreType.DMA,
        ],
        compiler_params=pltpu.CompilerParams(needs_layout_passes=False),
    )(_sev_body)
    sc_out = run(xt)

    tc_out = pl.pallas_call(
        _tc_body,
        grid=(TC_B // TC_W,),
        in_specs=[pl.BlockSpec(
            (NCONT, N, TC_W), lambda i: (0, 0, SC_B // TC_W + i))],
        out_specs=pl.BlockSpec((5, TC_W), lambda i: (0, i)),
        out_shape=jax.ShapeDtypeStruct((5, TC_B), jnp.float32),
    )(xt)

    out = jnp.concatenate([sc_out, tc_out], axis=1)
    return out.T
